# baseline (speedup 1.0000x reference)
import dataclasses
import functools

import jax
import jax.numpy as jnp
from jax import lax
from jax.experimental import pallas as pl
from jax.experimental.pallas import tpu as pltpu
from jax.experimental.pallas import tpu_sc as plsc

BATCH = 16384
FACTORS = 32
LANES = 16
NC = 2
NS = 16
NW = NC * NS
BPW = BATCH // NW
HALF = 256
CHUNK = 128


def _mf_body(uf4_hbm, if4_hbm, ui_hbm, ii_hbm, out_hbm,
             ui_raw, ii_raw, urow, irow, u4_v, v4_v, o_v, sem):
    wid = lax.axis_index("s") * NC + lax.axis_index("c")

    pltpu.sync_copy(ui_hbm.at[pl.ds(wid * 4, 4)], ui_raw)
    pltpu.sync_copy(ii_hbm.at[pl.ds(wid * 4, 4)], ii_raw)

    lane = lax.broadcasted_iota(jnp.int32, (LANES,), 0)

    for h in range(2):
        for j in range(2):
            for k in range(CHUNK // LANES):
                sl = pl.ds(k * LANES, LANES)
                urow[j, sl] = ui_raw[h * 2 + j, sl] >> 2
                irow[j, sl] = ii_raw[h * 2 + j, sl] >> 2
        copies = []
        for j in range(2):
            copies.append(pltpu.async_copy(
                uf4_hbm.at[urow.at[j]], u4_v.at[pl.ds(j * CHUNK, CHUNK)], sem))
            copies.append(pltpu.async_copy(
                if4_hbm.at[irow.at[j]], v4_v.at[pl.ds(j * CHUNK, CHUNK)], sem))
        for c in copies:
            c.wait()

        for g in range(HALF // LANES):
            sl = pl.ds((g % 8) * LANES, LANES)
            iu = ui_raw[h * 2 + g // 8, sl]
            iv = ii_raw[h * 2 + g // 8, sl]
            lu = (iu & 3) * FACTORS
            lv = (iv & 3) * FACTORS
            rowvec = lane + g * LANES
            acc = jnp.zeros((LANES,), jnp.float32)
            for f in range(FACTORS):
                gu = plsc.load_gather(u4_v, [rowvec, lu + f])
                gv = plsc.load_gather(v4_v, [rowvec, lv + f])
                acc = acc + gu * gv
            o_v[pl.ds(h * HALF + g * LANES, LANES)] = acc

    pltpu.sync_copy(o_v, out_hbm.at[pl.ds(wid * BPW, BPW)])


def kernel(user_idx, item_idx, user_factors, item_factors):
    uf4 = user_factors.reshape(250000, 128)
    if4 = item_factors.reshape(250000, 128)
    uidx = user_idx.astype(jnp.int32).reshape(128, 128)
    iidx = item_idx.astype(jnp.int32).reshape(128, 128)

    mesh = plsc.VectorSubcoreMesh(core_axis_name="c", subcore_axis_name="s")
    cp = pltpu.CompilerParams()
    if "needs_layout_passes" in pltpu.CompilerParams.__dataclass_fields__:
        cp = dataclasses.replace(cp, needs_layout_passes=False,
                                 use_tc_tiling_on_sc=True)
    mf = functools.partial(
        pl.kernel,
        compiler_params=cp,
        out_type=jax.ShapeDtypeStruct((BATCH,), jnp.float32),
        mesh=mesh,
        scratch_types=[
            pltpu.VMEM((4, CHUNK), jnp.int32),
            pltpu.VMEM((4, CHUNK), jnp.int32),
            pltpu.VMEM((2, CHUNK), jnp.int32),
            pltpu.VMEM((2, CHUNK), jnp.int32),
            pltpu.VMEM((HALF, 128), jnp.float32),
            pltpu.VMEM((HALF, 128), jnp.float32),
            pltpu.VMEM((BPW,), jnp.float32),
            pltpu.SemaphoreType.DMA,
        ],
    )(_mf_body)
    return mf(uf4, if4, uidx, iidx)


# speedup vs baseline: 4.1049x; 4.1049x over previous
import dataclasses
import functools

import jax
import jax.numpy as jnp
from jax import lax
from jax.experimental import pallas as pl
from jax.experimental.pallas import tpu as pltpu
from jax.experimental.pallas import tpu_sc as plsc

BATCH = 16384
FACTORS = 32
LANES = 16
NC = 2
NS = 16
NW = NC * NS
BPW = BATCH // NW
NU = 1000000
WC = 512
TAILFULL = 999424
TAILPART = 999936
SCAP = 720
WCAP = 80
TIER0 = 16
SROWS = BATCH + 2048
IDXCHUNK = 2048


def _sweep_table(tbl, idx_hbm, staged, wid, ichunk, rs, bs, wr, wb,
                 wbt0, wbt1, wbo, slab, tails, stage,
                 sem_i, sem_s0, sem_s1, sem_c0, sem_c1):
    lane = lax.broadcasted_iota(jnp.int32, (LANES,), 0)

    cnt = jnp.int32(0)
    for ch in range(BATCH // IDXCHUNK):
        pltpu.async_copy(
            idx_hbm.at[pl.ds(ch * IDXCHUNK, IDXCHUNK)], ichunk, sem_i).wait()

        def pf_body(i, cnt):
            r = ichunk[pl.ds(i * LANES, LANES)]
            m = ((r >> 9) & 31) == wid
            plsc.store_compressed(rs.at[pl.ds(cnt, LANES)], r, mask=m)
            b = lane + (i * LANES + ch * IDXCHUNK)
            plsc.store_compressed(bs.at[pl.ds(cnt, LANES)], b, mask=m)
            return cnt + jnp.sum(m.astype(jnp.int32))

        cnt = lax.fori_loop(0, IDXCHUNK // LANES, pf_body, cnt)
    nvec = (cnt + LANES - 1) >> 4

    def drain_scatter(p):
        pltpu.make_async_copy(
            stage.at[p, pl.ds(0, TIER0), pl.ds(0, 128)],
            staged.at[(wbt0 if p == 0 else wbt1).at[p]],
            sem_c0 if p == 0 else sem_c1).wait()

    def drain_slab(p):
        pltpu.make_async_copy(
            tbl.at[:, pl.ds(0, WC)], slab.at[p],
            sem_s0 if p == 0 else sem_s1).wait()

    def fire_slab(g, p):
        start = pl.multiple_of(wid * WC + g * 16384, 128)
        pltpu.async_copy(tbl.at[:, pl.ds(start, WC)], slab.at[p],
                         sem_s0 if p == 0 else sem_s1)

    def process_window(g, p, start, slab_ref):
        jglob = wid + 32 * g
        trash = BATCH + wid * 64 + g
        for k in range(WCAP // LANES):
            wr[p, pl.ds(k * LANES, LANES)] = jnp.full((LANES,), start,
                                                      jnp.int32)
            wb[p, pl.ds(k * LANES, LANES)] = jnp.full((LANES,), trash,
                                                      jnp.int32)

        def scan_body(i, wcnt):
            r = rs[pl.ds(i * LANES, LANES)]
            valid = (i * LANES + lane) < cnt
            m = ((r >> 9) == jglob) & valid
            plsc.store_compressed(wr.at[p, pl.ds(wcnt, LANES)], r, mask=m)
            b = bs[pl.ds(i * LANES, LANES)]
            plsc.store_compressed(wb.at[p, pl.ds(wcnt, LANES)], b, mask=m)
            return wcnt + jnp.sum(m.astype(jnp.int32))

        wcnt = lax.fori_loop(0, nvec, scan_body, jnp.int32(0))

        for k in range(WCAP // LANES):
            def grp(k=k):
                rloc = wr[p, pl.ds(k * LANES, LANES)] - start
                rows = lane + k * LANES

                @pl.loop(0, FACTORS)
                def _(f):
                    fv = jnp.zeros((LANES,), jnp.int32) + f
                    vals = plsc.load_gather(slab_ref, [fv, rloc])
                    plsc.store_scatter(stage.at[p], [rows, fv], vals)
            if k == 0:
                grp()
            else:
                pl.when(k * LANES < wcnt)(grp)

        wbt = wbt0 if p == 0 else wbt1
        wbt[p, pl.ds(0, LANES)] = wb[p, pl.ds(0, LANES)]
        pltpu.async_copy(stage.at[p, pl.ds(0, TIER0), pl.ds(0, 128)],
                         staged.at[wbt.at[p]],
                         sem_c0 if p == 0 else sem_c1)

        @pl.when(wcnt > TIER0)
        def _():
            for k in range(4):
                wbo[p, pl.ds(k * LANES, LANES)] = (
                    wb[p, pl.ds(TIER0 + k * LANES, LANES)])
            pltpu.sync_copy(stage.at[p, pl.ds(TIER0, 64), pl.ds(0, 128)],
                            staged.at[wbo.at[p]])

        return wcnt

    fire_slab(0, 0)
    fire_slab(1, 1)

    @pl.loop(0, 30)
    def _(i):
        g0 = i * 2
        for p in range(2):
            g = g0 + p

            @pl.when(g >= 2)
            def _():
                drain_scatter(p)

            drain_slab(p)
            start = wid * WC + g * 16384
            process_window(g, p, start, slab.at[p])

            @pl.when(g + 2 <= 60)
            def _():
                fire_slab(g + 2, p)

    drain_scatter(0)
    drain_slab(0)
    process_window(jnp.int32(60), 0, wid * WC + 60 * 16384, slab.at[0])

    @pl.when(wid == 0)
    def _():
        drain_scatter(1)
        pltpu.sync_copy(tbl.at[:, pl.ds(TAILFULL, WC)], slab.at[1])
        process_window(jnp.int32(61), 1, jnp.int32(TAILFULL), slab.at[1])

    @pl.when(wid == 1)
    def _():
        drain_scatter(1)
        pltpu.sync_copy(tbl.at[:, pl.ds(TAILPART, 64)], tails)
        process_window(jnp.int32(61), 1, jnp.int32(TAILPART), tails)

    drain_scatter(0)
    drain_scatter(1)


def _sweep_body(ut_hbm, it_hbm, ui_hbm, ii_hbm, su_hbm, sv_hbm,
                ichunk, rs_u, bs_u, rs_v, bs_v, wr, wb, wbt0, wbt1, wbo,
                slab, tails, stage, sem_i, sem_s0, sem_s1, sem_c0, sem_c1):
    wid = lax.axis_index("s") * NC + lax.axis_index("c")
    _sweep_table(ut_hbm, ui_hbm, su_hbm, wid, ichunk, rs_u, bs_u, wr, wb,
                 wbt0, wbt1, wbo, slab, tails, stage,
                 sem_i, sem_s0, sem_s1, sem_c0, sem_c1)
    _sweep_table(it_hbm, ii_hbm, sv_hbm, wid, ichunk, rs_v, bs_v, wr, wb,
                 wbt0, wbt1, wbo, slab, tails, stage,
                 sem_i, sem_s0, sem_s1, sem_c0, sem_c1)


def _pair_body(su_hbm, sv_hbm, out_hbm, u_v, v_v, o_v, sem):
    wid = lax.axis_index("s") * NC + lax.axis_index("c")
    base = wid * BPW
    lane = lax.broadcasted_iota(jnp.int32, (LANES,), 0)
    for h in range(2):
        pltpu.async_copy(su_hbm.at[pl.ds(base + h * 256, 256)], u_v,
                         sem).wait()
        pltpu.async_copy(sv_hbm.at[pl.ds(base + h * 256, 256)], v_v,
                         sem).wait()

        @pl.loop(0, 256 // LANES)
        def _(g):
            t = jnp.zeros((LANES,), jnp.float32)
            for k in range(LANES):
                r = g * LANES + k
                s = jnp.sum(
                    u_v[r, pl.ds(0, LANES)] * v_v[r, pl.ds(0, LANES)]
                    + u_v[r, pl.ds(LANES, LANES)]
                    * v_v[r, pl.ds(LANES, LANES)])
                t = jnp.where(lane == k, s, t)
            o_v[pl.ds(h * 256 + g * LANES, LANES)] = t

    pltpu.sync_copy(o_v, out_hbm.at[pl.ds(base, BPW)])


def kernel(user_idx, item_idx, user_factors, item_factors):
    uidx = user_idx.astype(jnp.int32)
    iidx = item_idx.astype(jnp.int32)

    mesh = plsc.VectorSubcoreMesh(core_axis_name="c", subcore_axis_name="s")
    cp = pltpu.CompilerParams()
    if "needs_layout_passes" in pltpu.CompilerParams.__dataclass_fields__:
        cp = dataclasses.replace(cp, needs_layout_passes=False,
                                 use_tc_tiling_on_sc=True)

    sweep = functools.partial(
        pl.kernel,
        compiler_params=cp,
        out_type=[jax.ShapeDtypeStruct((SROWS, 128), jnp.float32),
                  jax.ShapeDtypeStruct((SROWS, 128), jnp.float32)],
        mesh=mesh,
        scratch_types=[
            pltpu.VMEM((IDXCHUNK,), jnp.int32),
            pltpu.VMEM((SCAP,), jnp.int32),
            pltpu.VMEM((SCAP,), jnp.int32),
            pltpu.VMEM((SCAP,), jnp.int32),
            pltpu.VMEM((SCAP,), jnp.int32),
            pltpu.VMEM((2, WCAP + LANES), jnp.int32),
            pltpu.VMEM((2, WCAP + LANES), jnp.int32),
            pltpu.VMEM((2, TIER0), jnp.int32),
            pltpu.VMEM((2, TIER0), jnp.int32),
            pltpu.VMEM((2, 64), jnp.int32),
            pltpu.VMEM((2, FACTORS, WC), jnp.float32),
            pltpu.VMEM((FACTORS, 64), jnp.float32),
            pltpu.VMEM((2, WCAP, 136), jnp.float32),
            pltpu.SemaphoreType.DMA,
            pltpu.SemaphoreType.DMA,
            pltpu.SemaphoreType.DMA,
            pltpu.SemaphoreType.DMA,
            pltpu.SemaphoreType.DMA,
        ],
    )(_sweep_body)
    staged_u, staged_v = sweep(user_factors.T, item_factors.T, uidx, iidx)

    pair = functools.partial(
        pl.kernel,
        compiler_params=cp,
        out_type=jax.ShapeDtypeStruct((BATCH,), jnp.float32),
        mesh=mesh,
        scratch_types=[
            pltpu.VMEM((256, 128), jnp.float32),
            pltpu.VMEM((256, 128), jnp.float32),
            pltpu.VMEM((BPW,), jnp.float32),
            pltpu.SemaphoreType.DMA,
        ],
    )(_pair_body)
    return pair(staged_u, staged_v)
